# baseline (speedup 1.0000x reference)
import functools

import jax
import jax.numpy as jnp
from jax import lax
from jax.experimental import pallas as pl
from jax.experimental.pallas import tpu as pltpu
from jax.experimental.pallas import tpu_sc as plsc

B = 4096
S = 200
D = 64
C = 100
NC = 2
NS = 16
NW = NC * NS
BPW = B // NW
HALF = S // 2


def _issue(tbl, idx_v, buf_v, sem, row, par):
    pltpu.make_async_copy(
        tbl.at[idx_v.at[2 * row]], buf_v.at[par, pl.ds(0, HALF)], sem).start()
    pltpu.make_async_copy(
        tbl.at[idx_v.at[2 * row + 1]], buf_v.at[par, pl.ds(HALF, HALF)], sem).start()


def _wait(tbl, idx_v, buf_v, sem, row, par):
    pltpu.make_async_copy(
        tbl.at[idx_v.at[2 * row]], buf_v.at[par, pl.ds(0, HALF)], sem).wait()
    pltpu.make_async_copy(
        tbl.at[idx_v.at[2 * row + 1]], buf_v.at[par, pl.ds(HALF, HALF)], sem).wait()


def _accum_store(buf_v, pool_v, row, par):
    z = jnp.zeros((16,), jnp.float32)

    def body(i, accs):
        accs = list(accs)
        for u in range(2):
            for j in range(4):
                accs[j] = accs[j] + buf_v[par, 2 * i + u, pl.ds(16 * j, 16)]
        return tuple(accs)

    accs = lax.fori_loop(0, S // 2, body, (z, z, z, z))
    for j in range(4):
        pool_v[row, pl.ds(16 * j, 16)] = accs[j]


@functools.partial(
    pl.kernel,
    mesh=plsc.VectorSubcoreMesh(core_axis_name="c", subcore_axis_name="s"),
    out_type=jax.ShapeDtypeStruct((B, D), jnp.float32),
    scratch_types=[
        pltpu.VMEM((2 * BPW, HALF), jnp.int32),
        pltpu.VMEM((2, S, D), jnp.float32),
        pltpu.VMEM((BPW, D), jnp.float32),
        pltpu.SemaphoreType.DMA,
        pltpu.SemaphoreType.DMA,
    ],
    compiler_params=pltpu.CompilerParams(use_tc_tiling_on_sc=False),
)
def _sc_pool(x2_hbm, tbl_hbm, out_hbm, idx_v, buf_v, pool_v, sem0, sem1):
    wid = lax.axis_index("s") * NC + lax.axis_index("c")
    base = wid * BPW
    pltpu.sync_copy(x2_hbm.at[pl.ds(2 * base, 2 * BPW)], idx_v)
    _issue(tbl_hbm, idx_v, buf_v, sem0, 0, 0)

    def outer(g, carry):
        r0 = 2 * g
        _issue(tbl_hbm, idx_v, buf_v, sem1, r0 + 1, 1)
        _wait(tbl_hbm, idx_v, buf_v, sem0, r0, 0)
        _accum_store(buf_v, pool_v, r0, 0)

        @pl.when(r0 + 2 < BPW)
        def _():
            _issue(tbl_hbm, idx_v, buf_v, sem0, r0 + 2, 0)

        _wait(tbl_hbm, idx_v, buf_v, sem1, r0 + 1, 1)
        _accum_store(buf_v, pool_v, r0 + 1, 1)
        return carry

    lax.fori_loop(0, BPW // 2, outer, 0)
    pltpu.sync_copy(pool_v, out_hbm.at[pl.ds(base, BPW)])


def _head_body(p_ref, w_ref, b_ref, o_ref):
    o_ref[...] = lax.dot_general(
        p_ref[...], w_ref[...], (((1,), (1,)), ((), ())),
        preferred_element_type=jnp.float32) + b_ref[...]


def _tc_head(pooled, W, b2):
    BB = 1024
    return pl.pallas_call(
        _head_body,
        grid=(B // BB,),
        in_specs=[
            pl.BlockSpec((BB, D), lambda i: (i, 0)),
            pl.BlockSpec((C, D), lambda i: (0, 0)),
            pl.BlockSpec((1, C), lambda i: (0, 0)),
        ],
        out_specs=pl.BlockSpec((BB, C), lambda i: (i, 0)),
        out_shape=jax.ShapeDtypeStruct((B, C), jnp.float32),
    )(pooled, W, b2)


@jax.jit
def kernel(x, table, W, b):
    x2 = x.astype(jnp.int32).reshape(2 * B, HALF)
    pooled = _sc_pool(x2, table)
    return _tc_head(pooled, W, b.reshape(1, C))
